# baseline (speedup 1.0000x reference)
import functools

import jax
import jax.numpy as jnp
from jax import lax
from jax.experimental import pallas as pl
from jax.experimental.pallas import tpu as pltpu
from jax.experimental.pallas import tpu_sc as plsc

_B, _N, _F, _D = 128, 100, 64, 64
_G = 4
_GN = _G * _N
_NGROUPS = _B // _G
_UNROLL = 4
_CH = 16
_CHN = _CH * _N
_NCHUNK = _B // _CH
_NEDGE = 1500
_LANES = 16


def _counts_body(edges_hbm, out_hbm, src_v, dst_v, acc_v):
    c = lax.axis_index("c")
    s = lax.axis_index("s")

    @pl.when(jnp.logical_and(c == 0, s == 0))
    def _():
        pltpu.sync_copy(edges_hbm.at[0], src_v)
        pltpu.sync_copy(edges_hbm.at[1], dst_v)

        zeros = jnp.zeros((_LANES,), jnp.float32)

        def zero_body(i, carry):
            for u in range(25):
                acc_v[pl.ds((i * 25 + u) * _LANES, _LANES)] = zeros
            return carry

        lax.fori_loop(0, (_N * _N) // (_LANES * 25), zero_body, 0)

        ones = jnp.full((_LANES,), 1.0, jnp.float32)
        lane = lax.iota(jnp.int32, _LANES)

        def edge_body(i, carry):
            for u in range(2):
                j = i * 2 + u
                sv = src_v[pl.ds(j * _LANES, _LANES)]
                dv = dst_v[pl.ds(j * _LANES, _LANES)]
                valid = (j * _LANES + lane) < _NEDGE
                plsc.addupdate_scatter(acc_v, [dv * _N + sv], ones,
                                       mask=valid)
            return carry

        lax.fori_loop(0, (_NEDGE + 2 * _LANES - 1) // (2 * _LANES),
                      edge_body, 0)

        def loop_body(i, carry):
            node = i * _LANES + lane
            valid = node < _N
            plsc.addupdate_scatter(acc_v, [node * (_N + 1)], ones, mask=valid)
            return carry

        lax.fori_loop(0, (_N + _LANES - 1) // _LANES, loop_body, 0)
        pltpu.sync_copy(acc_v, out_hbm)


_build_counts = functools.partial(
    pl.kernel,
    mesh=plsc.VectorSubcoreMesh(core_axis_name="c", subcore_axis_name="s"),
    out_type=jax.ShapeDtypeStruct((_N * _N,), jnp.float32),
    compiler_params=pltpu.CompilerParams(needs_layout_passes=False),
    scratch_types=[
        pltpu.VMEM((_NEDGE + 4,), jnp.int32),
        pltpu.VMEM((_NEDGE + 4,), jnp.int32),
        pltpu.VMEM((_N * _N,), jnp.float32),
    ],
)(_counts_body)


def _mm(a, b, ca, cb):
    return lax.dot_general(a, b, (((ca,), (cb,)), ((), ())),
                           precision=lax.Precision.HIGHEST,
                           preferred_element_type=jnp.float32)


def _node_mask(rows):
    r_id = lax.broadcasted_iota(jnp.int32, (rows, _N), 0)
    c_id = lax.broadcasted_iota(jnp.int32, (rows, _N), 1)
    return (lax.rem(r_id, _N) == c_id).astype(jnp.float32)


def _stripe_mask(nb, rows):
    q_id = lax.broadcasted_iota(jnp.int32, (nb, rows), 0)
    r_id = lax.broadcasted_iota(jnp.int32, (nb, rows), 1)
    return ((r_id // _N) == q_id).astype(jnp.float32)


def _fused_body(x_ref, emb_ref, c_ref, wlin_ref, attij_ref, attemij_ref,
                bias_ref, g1_ref, b1_ref, g2_ref, b2_ref, ow_ref,
                obn_ref, o_ref, out_s):
    wlin = wlin_ref[:]
    attij = attij_ref[:]
    bias = bias_ref[:]

    nm16 = _node_mask(_CHN)
    nm4 = nm16[:_GN, :]
    sel4 = _stripe_mask(_G, _GN)
    sel16 = _stripe_mask(_CH, _CHN)

    emb16 = _mm(nm16, emb_ref[:], 1, 0)
    emb4 = emb16[:_GN, :]
    c4 = _mm(nm4, c_ref[:], 1, 0)
    eij = _mm(emb4, attemij_ref[:], 1, 1)
    ei = eij[:, 0:1]
    ej = eij[:, 1:2]

    zstat = jnp.zeros((1, _D), jnp.float32)

    def _one_group(gidx):
        wx_g = _mm(x_ref[pl.ds(gidx * _GN, _GN), :], wlin, 1, 1)
        aiaj = _mm(wx_g, attij, 1, 1)
        ai = aiaj[:, 0:1] + ei
        aj_c = aiaj[:, 1:2] + ej
        ajg = _mm(sel4, aj_c * nm4, 1, 0)
        aj_rep = _mm(sel4, ajg, 0, 0)
        alpha = ai + aj_rep
        alpha = jnp.where(alpha >= 0, alpha, 0.2 * alpha)
        am = jnp.max(jnp.where(c4 > 0, alpha, -1e30), axis=1, keepdims=True)
        ex = c4 * jnp.exp(jnp.minimum(alpha - am, 0.0))
        den = jnp.sum(ex, axis=1, keepdims=True)
        w = ex / (den + 1e-16)
        return wx_g, w

    def group_body(g, carry):
        s1, s2 = carry
        parts = [_one_group(_UNROLL * g + u) for u in range(_UNROLL)]
        for u, (wx_g, w) in enumerate(parts):
            base = (_UNROLL * g + u) * _GN
            out_g = jnp.concatenate(
                [_mm(w[q * _N:(q + 1) * _N, :],
                     wx_g[q * _N:(q + 1) * _N, :], 1, 0) for q in range(_G)],
                axis=0) + bias
            out_s[pl.ds(base, _GN), :] = out_g
            s1 = s1 + jnp.sum(out_g, axis=0, keepdims=True)
            s2 = s2 + jnp.sum(out_g * out_g, axis=0, keepdims=True)
        return s1, s2

    s1, s2 = lax.fori_loop(0, _NGROUPS // _UNROLL, group_body, (zstat, zstat))

    n_rows = float(_B * _N)
    mu1 = s1 / n_rows
    var1 = s2 / n_rows - mu1 * mu1
    scale1 = lax.rsqrt(var1 + 1e-5) * g1_ref[:]
    b1 = b1_ref[:]

    def bn1_body(i, carry):
        t1, t2 = carry
        base = i * _CHN
        o1 = (out_s[pl.ds(base, _CHN), :] - mu1) * scale1 + b1
        h = jnp.maximum(o1, 0.0) * emb16
        out_s[pl.ds(base, _CHN), :] = h
        t1 = t1 + jnp.sum(h, axis=0, keepdims=True)
        t2 = t2 + jnp.sum(h * h, axis=0, keepdims=True)
        return t1, t2

    t1, t2 = lax.fori_loop(0, _NCHUNK, bn1_body, (zstat, zstat))

    mu2 = t1 / n_rows
    var2 = t2 / n_rows - mu2 * mu2
    scale2 = lax.rsqrt(var2 + 1e-5) * g2_ref[:]
    b2 = b2_ref[:]
    ow = ow_ref[:]
    obn = obn_ref[:]

    def bn2_body(i, carry):
        base = i * _CHN
        h2 = jnp.maximum((out_s[pl.ds(base, _CHN), :] - mu2) * scale2 + b2,
                         0.0)
        oc = _mm(h2, ow, 1, 1)
        orows = _mm(sel16, oc * nm16, 1, 0)
        o_ref[pl.ds(i * _CH, _CH), :] = orows + obn
        return carry

    lax.fori_loop(0, _NCHUNK, bn2_body, 0)


_fused = pl.pallas_call(
    _fused_body,
    out_shape=jax.ShapeDtypeStruct((_B, _N), jnp.float32),
    scratch_shapes=[
        pltpu.VMEM((_B * _N, _D), jnp.float32),
    ],
)


def kernel(data, labels, org_edge_index, emb_table, W_lin, att_i, att_j,
           att_em_i, att_em_j, gnn_bias, bn1_gamma, bn1_beta, bn2_gamma,
           bn2_beta, out_W, out_b):
    B, N, F = data.shape

    edges = jnp.pad(org_edge_index, ((0, 0), (0, 4)))
    counts = _build_counts(edges)
    c_mat = counts.reshape(N, N)

    attij = jnp.stack([att_i, att_j])
    attemij = jnp.stack([att_em_i, att_em_j])
    obn = jnp.broadcast_to(out_b.reshape(1, 1), (1, N))

    return _fused(data.reshape(B * N, F), emb_table, c_mat, W_lin, attij,
                  attemij,
                  gnn_bias.reshape(1, -1), bn1_gamma.reshape(1, -1),
                  bn1_beta.reshape(1, -1), bn2_gamma.reshape(1, -1),
                  bn2_beta.reshape(1, -1), out_W.reshape(1, -1), obn)


# speedup vs baseline: 1.1741x; 1.1741x over previous
import functools

import jax
import jax.numpy as jnp
from jax import lax
from jax.experimental import pallas as pl
from jax.experimental.pallas import tpu as pltpu
from jax.experimental.pallas import tpu_sc as plsc

_B, _N, _F, _D = 128, 100, 64, 64
_G = 4
_GN = _G * _N
_NGROUPS = _B // _G
_UNROLL = 8
_CH = 16
_CHN = _CH * _N
_NCHUNK = _B // _CH
_NEDGE = 1500
_LANES = 16


def _counts_body(edges_hbm, out_hbm, src_v, dst_v, acc_v):
    c = lax.axis_index("c")
    s = lax.axis_index("s")

    @pl.when(jnp.logical_and(c == 0, s == 0))
    def _():
        pltpu.sync_copy(edges_hbm.at[0], src_v)
        pltpu.sync_copy(edges_hbm.at[1], dst_v)

        zeros = jnp.zeros((_LANES,), jnp.float32)

        def zero_body(i, carry):
            for u in range(25):
                acc_v[pl.ds((i * 25 + u) * _LANES, _LANES)] = zeros
            return carry

        lax.fori_loop(0, (_N * _N) // (_LANES * 25), zero_body, 0)

        ones = jnp.full((_LANES,), 1.0, jnp.float32)
        lane = lax.iota(jnp.int32, _LANES)

        def edge_body(i, carry):
            for u in range(2):
                j = i * 2 + u
                sv = src_v[pl.ds(j * _LANES, _LANES)]
                dv = dst_v[pl.ds(j * _LANES, _LANES)]
                valid = (j * _LANES + lane) < _NEDGE
                plsc.addupdate_scatter(acc_v, [dv * _N + sv], ones,
                                       mask=valid)
            return carry

        lax.fori_loop(0, (_NEDGE + 2 * _LANES - 1) // (2 * _LANES),
                      edge_body, 0)

        def loop_body(i, carry):
            node = i * _LANES + lane
            valid = node < _N
            plsc.addupdate_scatter(acc_v, [node * (_N + 1)], ones, mask=valid)
            return carry

        lax.fori_loop(0, (_N + _LANES - 1) // _LANES, loop_body, 0)
        pltpu.sync_copy(acc_v, out_hbm)


_build_counts = functools.partial(
    pl.kernel,
    mesh=plsc.VectorSubcoreMesh(core_axis_name="c", subcore_axis_name="s"),
    out_type=jax.ShapeDtypeStruct((_N * _N,), jnp.float32),
    compiler_params=pltpu.CompilerParams(needs_layout_passes=False),
    scratch_types=[
        pltpu.VMEM((_NEDGE + 4,), jnp.int32),
        pltpu.VMEM((_NEDGE + 4,), jnp.int32),
        pltpu.VMEM((_N * _N,), jnp.float32),
    ],
)(_counts_body)


def _mm(a, b, ca, cb):
    return lax.dot_general(a, b, (((ca,), (cb,)), ((), ())),
                           precision=lax.Precision.HIGHEST,
                           preferred_element_type=jnp.float32)


def _node_mask(rows):
    r_id = lax.broadcasted_iota(jnp.int32, (rows, _N), 0)
    c_id = lax.broadcasted_iota(jnp.int32, (rows, _N), 1)
    return (lax.rem(r_id, _N) == c_id).astype(jnp.float32)


def _stripe_mask(nb, rows):
    q_id = lax.broadcasted_iota(jnp.int32, (nb, rows), 0)
    r_id = lax.broadcasted_iota(jnp.int32, (nb, rows), 1)
    return ((r_id // _N) == q_id).astype(jnp.float32)


def _fused_body(x_ref, emb_ref, c_ref, wlin_ref, ai_ref, aj_ref, aei_ref,
                aej_ref, bias_ref, g1_ref, b1_ref, g2_ref, b2_ref, ow_ref,
                obn_ref, o_ref, out_s):
    wlin = wlin_ref[:]
    att_i = ai_ref[:]
    att_j = aj_ref[:]
    bias = bias_ref[:]

    nm16 = _node_mask(_CHN)
    nm4 = nm16[:_GN, :]
    sel4 = _stripe_mask(_G, _GN)
    sel16 = _stripe_mask(_CH, _CHN)

    emb16 = _mm(nm16, emb_ref[:], 1, 0)
    emb4 = emb16[:_GN, :]
    c4 = _mm(nm4, c_ref[:], 1, 0)
    ei = _mm(emb4, aei_ref[:], 1, 1)
    ej = _mm(emb4, aej_ref[:], 1, 1)

    zstat = jnp.zeros((1, _D), jnp.float32)

    def _one_group(gidx):
        wx_g = _mm(x_ref[pl.ds(gidx * _GN, _GN), :], wlin, 1, 1)
        ai = _mm(wx_g, att_i, 1, 1) + ei
        aj_c = _mm(wx_g, att_j, 1, 1) + ej
        ajg = _mm(sel4, aj_c * nm4, 1, 0)
        aj_rep = _mm(sel4, ajg, 0, 0)
        alpha = ai + aj_rep
        alpha = jnp.where(alpha >= 0, alpha, 0.2 * alpha)
        am = jnp.max(jnp.where(c4 > 0, alpha, -1e30), axis=1, keepdims=True)
        ex = c4 * jnp.exp(jnp.minimum(alpha - am, 0.0))
        den = jnp.sum(ex, axis=1, keepdims=True)
        w = ex / (den + 1e-16)
        return wx_g, w

    def group_body(g, carry):
        s1, s2 = carry
        parts = [_one_group(_UNROLL * g + u) for u in range(_UNROLL)]
        for u, (wx_g, w) in enumerate(parts):
            base = (_UNROLL * g + u) * _GN
            out_g = jnp.concatenate(
                [_mm(w[q * _N:(q + 1) * _N, :],
                     wx_g[q * _N:(q + 1) * _N, :], 1, 0) for q in range(_G)],
                axis=0) + bias
            out_s[pl.ds(base, _GN), :] = out_g
            s1 = s1 + jnp.sum(out_g, axis=0, keepdims=True)
            s2 = s2 + jnp.sum(out_g * out_g, axis=0, keepdims=True)
        return s1, s2

    s1, s2 = lax.fori_loop(0, _NGROUPS // _UNROLL, group_body, (zstat, zstat))

    n_rows = float(_B * _N)
    mu1 = s1 / n_rows
    var1 = s2 / n_rows - mu1 * mu1
    scale1 = lax.rsqrt(var1 + 1e-5) * g1_ref[:]
    b1 = b1_ref[:]

    def bn1_body(i, carry):
        t1, t2 = carry
        base = i * _CHN
        o1 = (out_s[pl.ds(base, _CHN), :] - mu1) * scale1 + b1
        h = jnp.maximum(o1, 0.0) * emb16
        out_s[pl.ds(base, _CHN), :] = h
        t1 = t1 + jnp.sum(h, axis=0, keepdims=True)
        t2 = t2 + jnp.sum(h * h, axis=0, keepdims=True)
        return t1, t2

    t1, t2 = lax.fori_loop(0, _NCHUNK, bn1_body, (zstat, zstat))

    mu2 = t1 / n_rows
    var2 = t2 / n_rows - mu2 * mu2
    scale2 = lax.rsqrt(var2 + 1e-5) * g2_ref[:]
    b2 = b2_ref[:]
    ow = ow_ref[:]
    obn = obn_ref[:]

    def bn2_body(i, carry):
        base = i * _CHN
        h2 = jnp.maximum((out_s[pl.ds(base, _CHN), :] - mu2) * scale2 + b2,
                         0.0)
        oc = _mm(h2, ow, 1, 1)
        orows = _mm(sel16, oc * nm16, 1, 0)
        o_ref[pl.ds(i * _CH, _CH), :] = orows + obn
        return carry

    lax.fori_loop(0, _NCHUNK, bn2_body, 0)


_fused = pl.pallas_call(
    _fused_body,
    out_shape=jax.ShapeDtypeStruct((_B, _N), jnp.float32),
    scratch_shapes=[
        pltpu.VMEM((_B * _N, _D), jnp.float32),
    ],
)


def kernel(data, labels, org_edge_index, emb_table, W_lin, att_i, att_j,
           att_em_i, att_em_j, gnn_bias, bn1_gamma, bn1_beta, bn2_gamma,
           bn2_beta, out_W, out_b):
    B, N, F = data.shape

    edges = jnp.pad(org_edge_index, ((0, 0), (0, 4)))
    counts = _build_counts(edges)
    c_mat = counts.reshape(N, N)

    obn = jnp.broadcast_to(out_b.reshape(1, 1), (1, N))

    return _fused(data.reshape(B * N, F), emb_table, c_mat, W_lin,
                  att_i.reshape(1, -1), att_j.reshape(1, -1),
                  att_em_i.reshape(1, -1), att_em_j.reshape(1, -1),
                  gnn_bias.reshape(1, -1), bn1_gamma.reshape(1, -1),
                  bn1_beta.reshape(1, -1), bn2_gamma.reshape(1, -1),
                  bn2_beta.reshape(1, -1), out_W.reshape(1, -1), obn)
